# baseline (speedup 1.0000x reference)
import functools

import jax
import jax.numpy as jnp
from jax import lax
from jax.experimental import pallas as pl
from jax.experimental.pallas import tpu as pltpu
from jax.experimental.pallas import tpu_sc as plsc

NC = 2
NS = 16
L = 16
NW = NC * NS
CH = 128


def _gmf_body(n_chunks, d, uidx_hbm, iidx_hbm, utab, itab, wb_hbm, out_hbm,
              idx_u, idx_i, rows_u, rows_i, wb_v, out_v, sem):
  wid = lax.axis_index("s") * NC + lax.axis_index("c")
  b_per_w = n_chunks * CH

  pltpu.sync_copy(uidx_hbm.at[wid], idx_u)
  pltpu.sync_copy(iidx_hbm.at[wid], idx_i)
  pltpu.sync_copy(wb_hbm, wb_v)

  copies = []
  for j in range(n_chunks):
    copies.append(
        pltpu.async_copy(utab.at[idx_u.at[j]], rows_u.at[pl.ds(j * CH, CH)],
                         sem))
    copies.append(
        pltpu.async_copy(itab.at[idx_i.at[j]], rows_i.at[pl.ds(j * CH, CH)],
                         sem))
  for c in copies:
    c.wait()

  lanes = lax.iota(jnp.int32, L)
  bias = wb_v[pl.ds(d * L, L)]

  def group(g, _):
    rid = g * L + lanes

    def feat(f, acc):
      col = jnp.full((L,), f, jnp.int32)
      uvec = plsc.load_gather(rows_u, [rid, col])
      ivec = plsc.load_gather(rows_i, [rid, col])
      wvec = wb_v[pl.ds(f * L, L)]
      return acc + uvec * ivec * wvec

    acc = lax.fori_loop(0, d, feat, jnp.zeros((L,), jnp.float32))
    logits = acc + bias
    out_v[pl.ds(g * L, L)] = 1.0 / (1.0 + jnp.exp(-logits))
    return 0

  lax.fori_loop(0, b_per_w // L, group, 0)
  pltpu.sync_copy(out_v, out_hbm.at[pl.ds(wid * b_per_w, b_per_w)])


def kernel(user_indices, item_indices, user_table, item_table, W, b):
  batch = user_indices.shape[0]
  d = user_table.shape[1]
  b_per_w = batch // NW
  n_chunks = b_per_w // CH

  uidx3 = user_indices.astype(jnp.int32).reshape(NW, n_chunks, CH)
  iidx3 = item_indices.astype(jnp.int32).reshape(NW, n_chunks, CH)
  wb = jnp.repeat(jnp.concatenate([W[:, 0], b]), L).astype(jnp.float32)

  mesh = plsc.VectorSubcoreMesh(
      core_axis_name="c", subcore_axis_name="s", num_cores=NC, num_subcores=NS)
  run = pl.kernel(
      functools.partial(_gmf_body, n_chunks, d),
      out_type=jax.ShapeDtypeStruct((batch,), jnp.float32),
      mesh=mesh,
      compiler_params=pltpu.CompilerParams(
          needs_layout_passes=False, use_tc_tiling_on_sc=False),
      scratch_types=[
          pltpu.VMEM((n_chunks, CH), jnp.int32),
          pltpu.VMEM((n_chunks, CH), jnp.int32),
          pltpu.VMEM((b_per_w, d), jnp.float32),
          pltpu.VMEM((b_per_w, d), jnp.float32),
          pltpu.VMEM(((d + 1) * L,), jnp.float32),
          pltpu.VMEM((b_per_w,), jnp.float32),
          pltpu.SemaphoreType.DMA,
      ],
  )
  return run(uidx3, iidx3, user_table, item_table, wb)
